# Initial kernel scaffold; baseline (speedup 1.0000x reference)
import jax
import jax.numpy as jnp
from jax.experimental import pallas as pl


def kernel(opinions, weights):
    raise NotImplementedError("write your pallas kernel here")


# speedup vs baseline: 1.1821x; 1.1821x over previous
import functools

import jax
import jax.numpy as jnp
from jax import lax
from jax.experimental import pallas as pl
from jax.experimental.pallas import tpu as pltpu
from jax.experimental.pallas import tpu_sc as plsc

_NC = 2
_NS = 16
_L = 16
_NW = _NC * _NS


def kernel(opinions, weights):
    E, B, d = opinions.shape
    op_cat = opinions.reshape(E * B, d)
    b_per_w = B // _NW
    CH = 64
    n_ch = b_per_w // CH
    n_grp = b_per_w // _L

    mesh = plsc.VectorSubcoreMesh(core_axis_name="c", subcore_axis_name="s")

    @functools.partial(
        pl.kernel,
        out_type=jax.ShapeDtypeStruct((B, d), jnp.float32),
        mesh=mesh,
        scratch_types=[
            pltpu.VMEM((b_per_w * E,), jnp.float32),
            pltpu.VMEM((b_per_w,), jnp.int32),
            pltpu.VMEM((CH, d), jnp.float32),
            pltpu.SemaphoreType.DMA,
        ],
        compiler_params=pltpu.CompilerParams(needs_layout_passes=False),
    )
    def k(op_hbm, w_hbm, out_hbm, w_v, idx_v, rows_v, sem):
        wid = lax.axis_index("s") * _NC + lax.axis_index("c")
        base = wid * b_per_w

        pltpu.sync_copy(w_hbm.at[pl.ds(base * E, b_per_w * E)], w_v)

        def argmax_group(g, _):
            fvec = (g * _L + lax.iota(jnp.int32, _L)) * E
            best_v = plsc.load_gather(w_v, [fvec])
            best_i = jnp.zeros((_L,), jnp.int32)
            for e in range(1, E):
                v = plsc.load_gather(w_v, [fvec + e])
                p = v > best_v
                best_v = jnp.where(p, v, best_v)
                best_i = jnp.where(p, e, best_i)
            idx_v[pl.ds(g * _L, _L)] = best_i
            return 0

        lax.fori_loop(0, n_grp, argmax_group, 0)

        def gather_chunk(c, _):
            pltpu.async_copy(
                op_hbm.at[idx_v.at[pl.ds(c * CH, CH)]], rows_v, sem).wait()
            pltpu.sync_copy(rows_v, out_hbm.at[pl.ds(base + c * CH, CH)])
            return 0

        lax.fori_loop(0, n_ch, gather_chunk, 0)

    return k(op_cat, weights.reshape(B * E))


# speedup vs baseline: 1.2030x; 1.0176x over previous
import functools

import jax
import jax.numpy as jnp
from jax import lax
from jax.experimental import pallas as pl
from jax.experimental.pallas import tpu as pltpu
from jax.experimental.pallas import tpu_sc as plsc

_NC = 2
_NS = 16
_L = 16
_NW = _NC * _NS


def kernel(opinions, weights):
    E, B, d = opinions.shape
    op_cat = opinions.reshape(E * B, d)
    b_per_w = B // _NW
    CH = 64
    n_ch = b_per_w // CH
    n_grp = b_per_w // _L

    mesh = plsc.VectorSubcoreMesh(core_axis_name="c", subcore_axis_name="s")

    @functools.partial(
        pl.kernel,
        out_type=jax.ShapeDtypeStruct((B, d), jnp.float32),
        mesh=mesh,
        scratch_types=[
            pltpu.VMEM((b_per_w * E,), jnp.float32),
            pltpu.VMEM((b_per_w,), jnp.int32),
            pltpu.VMEM((2, CH, d), jnp.float32),
            pltpu.SemaphoreType.DMA,
            pltpu.SemaphoreType.DMA,
        ],
        compiler_params=pltpu.CompilerParams(needs_layout_passes=False),
    )
    def k(op_hbm, w_hbm, out_hbm, w_v, idx_v, rows_v, gsem, wsem):
        wid = lax.axis_index("s") * _NC + lax.axis_index("c")
        base = wid * b_per_w

        pltpu.sync_copy(w_hbm.at[pl.ds(base * E, b_per_w * E)], w_v)

        def argmax_group(g, _):
            fvec = (g * _L + lax.iota(jnp.int32, _L)) * E
            best_v = plsc.load_gather(w_v, [fvec])
            best_i = jnp.zeros((_L,), jnp.int32)
            for e in range(1, E):
                v = plsc.load_gather(w_v, [fvec + e])
                p = v > best_v
                best_v = jnp.where(p, v, best_v)
                best_i = jnp.where(p, e, best_i)
            idx_v[pl.ds(g * _L, _L)] = best_i
            return 0

        lax.fori_loop(0, n_grp, argmax_group, 0)

        def start_gather(c, b):
            return pltpu.async_copy(
                op_hbm.at[idx_v.at[pl.ds(c * CH, CH)]], rows_v.at[b], gsem)

        writes = [None, None]
        gets = [None, None]
        gets[0] = start_gather(0, 0)
        for c in range(n_ch):
            b = c & 1
            nb = b ^ 1
            gets[b].wait()
            if c + 1 < n_ch:
                if writes[nb] is not None:
                    writes[nb].wait()
                gets[nb] = start_gather(c + 1, nb)
            writes[b] = pltpu.async_copy(
                rows_v.at[b], out_hbm.at[pl.ds(base + c * CH, CH)], wsem)
        for w in writes:
            if w is not None:
                w.wait()

    return k(op_cat, weights.reshape(B * E))
